# baseline (speedup 1.0000x reference)
import functools

import jax
import jax.numpy as jnp
from jax import lax
from jax.experimental import pallas as pl
from jax.experimental.pallas import tpu as pltpu
from jax.experimental.pallas import tpu_sc as plsc

_NC = 2
_NS = 16
_NW = _NC * _NS
_L = 16

_CHUNK_ROWS = 8


@functools.cache
def _make_sc_add(B, S, D):
    seq_w = S // _NW
    C = _CHUNK_ROWS
    nch = seq_w // C
    nj = D // _L

    mesh = plsc.VectorSubcoreMesh(
        core_axis_name="c", subcore_axis_name="s",
        num_cores=_NC, num_subcores=_NS)

    def body(x_hbm, t_hbm, o_hbm, *scr):
        bufs, sems = scr[:18], scr[18:]
        xb = tuple(tuple(bufs[2 * b + d] for d in (0, 1)) for b in range(B))
        ob = tuple(tuple(bufs[8 + 2 * b + d] for d in (0, 1)) for b in range(B))
        tb = (bufs[16], bufs[17])
        slx = tuple(tuple(sems[2 * b + d] for d in (0, 1)) for b in range(B))
        sst = tuple(tuple(sems[8 + 2 * b + d] for d in (0, 1)) for b in range(B))
        slt = (sems[16], sems[17])

        wid = lax.axis_index("s") * _NC + lax.axis_index("c")
        r0 = wid * seq_w

        def start_load_t(c, d):
            pltpu.async_copy(t_hbm.at[pl.ds(r0 + c * C, C), :], tb[d], slt[d])

        def wait_load_t(d):
            pltpu.make_async_copy(t_hbm.at[pl.ds(0, C), :], tb[d], slt[d]).wait()

        def start_load_x(c, b, d):
            pltpu.async_copy(
                x_hbm.at[b, pl.ds(r0 + c * C, C), :], xb[b][d], slx[b][d])

        def wait_load_x(b, d):
            pltpu.make_async_copy(
                x_hbm.at[0, pl.ds(0, C), :], xb[b][d], slx[b][d]).wait()

        def wait_store(b, d):
            pltpu.make_async_copy(
                ob[b][d], o_hbm.at[0, pl.ds(0, C), :], sst[b][d]).wait()

        for d in (0, 1):
            start_load_t(d, d)
            for b in range(B):
                start_load_x(d, b, d)

        @pl.loop(0, nch, step=2)
        def _chunks(c):
            for d in (0, 1):
                cc = c + d
                wait_load_t(d)
                for b in range(B):
                    @pl.when(cc >= 2)
                    def _():
                        wait_store(b, d)

                    wait_load_x(b, d)

                t_ = tb[d]
                x_ = tuple(xb[b][d] for b in range(B))
                o_ = tuple(ob[b][d] for b in range(B))

                @plsc.parallel_loop(0, C, step=1, unroll=2)
                def _add(r):
                    for j in range(nj):
                        sl = pl.ds(j * _L, _L)
                        vt = t_[r, sl]
                        for b in range(B):
                            o_[b][r, sl] = x_[b][r, sl] + vt

                r = r0 + cc * C
                for b in range(B):
                    pltpu.async_copy(
                        ob[b][d], o_hbm.at[b, pl.ds(r, C), :], sst[b][d])

                @pl.when(cc + 2 < nch)
                def _():
                    start_load_t(cc + 2, d)
                    for b in range(B):
                        start_load_x(cc + 2, b, d)

        for b in range(B):
            for d in (0, 1):
                wait_store(b, d)

    f32 = jnp.float32
    return pl.kernel(
        body,
        out_type=jax.ShapeDtypeStruct((B, S, D), f32),
        mesh=mesh,
        scratch_types=(
            [pltpu.VMEM((C, D), f32)] * 18
            + [pltpu.SemaphoreType.DMA] * 18
        ),
        compiler_params=pltpu.CompilerParams(use_tc_tiling_on_sc=True),
    )


def kernel(x, pos_emb_table):
    B, S, D = x.shape
    return _make_sc_add(B, S, D)(x, pos_emb_table)
